# baseline (speedup 1.0000x reference)
import jax
import jax.numpy as jnp
from jax.experimental import pallas as pl
from jax.experimental.pallas import tpu as pltpu

K_Q = 65536
D = 128
B_K = 4096
MASK = K_Q - 1
R = 1024
NBLK = K_Q // R


def _select(s, p, r0, kpad_ref, queue_ref, out_ref):
    src = kpad_ref[pl.ds(s, R), :]
    rows = jax.lax.broadcasted_iota(jnp.int32, (R, D), 0) + r0
    in_win = ((rows - p) & MASK) < B_K
    out_ref[...] = jnp.where(in_win, src, queue_ref[...])


def _body(ptr_s, kpad_ref, queue_ref, out_ref):
    i = pl.program_id(0)
    r0 = i * R
    p = ptr_s[0]
    i0 = (r0 - p) & MASK
    i1 = (r0 + R - 1 - p) & MASK
    hit = jnp.logical_or(i0 < B_K, i1 < B_K)
    s = jnp.where(i0 < B_K, R + i0, jnp.maximum(0, i0 - (K_Q - R)))

    @pl.when(jnp.logical_not(hit))
    def _():
        out_ref[...] = queue_ref[...]

    @pl.when(jnp.logical_and(hit, (s & 7) == 0))
    def _():
        _select(pl.multiple_of(s, 8), p, r0, kpad_ref, queue_ref, out_ref)

    @pl.when(jnp.logical_and(hit, (s & 7) != 0))
    def _():
        _select(s, p, r0, kpad_ref, queue_ref, out_ref)


def kernel(keys, queue, ptr):
    pad = jnp.zeros((R, D), dtype=keys.dtype)
    kpad = jnp.concatenate([pad, keys, pad], axis=0)
    grid_spec = pltpu.PrefetchScalarGridSpec(
        num_scalar_prefetch=1,
        grid=(NBLK,),
        in_specs=[
            pl.BlockSpec((B_K + 2 * R, D), lambda i, p: (0, 0)),
            pl.BlockSpec((R, D), lambda i, p: (i, 0)),
        ],
        out_specs=pl.BlockSpec((R, D), lambda i, p: (i, 0)),
    )
    return pl.pallas_call(
        _body,
        grid_spec=grid_spec,
        out_shape=jax.ShapeDtypeStruct((K_Q, D), queue.dtype),
        compiler_params=pltpu.CompilerParams(
            dimension_semantics=("arbitrary",),
        ),
    )(ptr.astype(jnp.int32), kpad, queue)


# speedup vs baseline: 1.1799x; 1.1799x over previous
import functools

import jax
import jax.numpy as jnp
from jax import lax
from jax.experimental import pallas as pl
from jax.experimental.pallas import tpu as pltpu
from jax.experimental.pallas import tpu_sc as plsc

K_Q = 65536
D = 128
B_K = 4096
MASK = K_Q - 1
NC, NS = 2, 16
NW = NC * NS
SLAB = K_Q // NW
CH = 256
NCH = SLAB // CH
PC = 128


NBUF = 3


def _fast_slab(src_hbm, out_hbm, s0, d0, bufs, gsems, wsems):
    gh = [None] * NCH
    wh = [None] * NCH
    for i in range(min(NBUF, NCH)):
        gh[i] = pltpu.async_copy(src_hbm.at[pl.ds(s0 + i * CH, CH)],
                                 bufs[i], gsems[i])
    for i in range(NCH):
        b = i % NBUF
        gh[i].wait()
        wh[i] = pltpu.async_copy(bufs[b], out_hbm.at[pl.ds(d0 + i * CH, CH)],
                                 wsems[b])
        if i + NBUF < NCH:
            wh[i].wait()
            gh[i + NBUF] = pltpu.async_copy(
                src_hbm.at[pl.ds(s0 + (i + NBUF) * CH, CH)], bufs[b],
                gsems[b])
    for i in range(max(0, NCH - NBUF), NCH):
        wh[i].wait()


def _sc_body(keys_hbm, queue_hbm, ptr_hbm, out_hbm,
             ptr_v, idx_v, buf_v, buf_a, buf_b, buf_c,
             sga, sgb, sgc, swa, swb, swc):
    w = lax.axis_index("s") * NC + lax.axis_index("c")
    t0 = pl.multiple_of(w * SLAB, SLAB)
    pltpu.sync_copy(ptr_hbm, ptr_v.at[pl.ds(0, 1)])
    p = ptr_v[...][0]
    d0 = (p + t0) & MASK
    aligned = (p & 7) == 0
    fast = jnp.logical_and(aligned, d0 <= K_Q - SLAB)
    is_keys = t0 < B_K
    bufs = (buf_a, buf_b, buf_c)
    gsems, wsems = (sga, sgb, sgc), (swa, swb, swc)

    @pl.when(jnp.logical_and(fast, is_keys))
    def _():
        d0a = pl.multiple_of(d0, 8)
        _fast_slab(keys_hbm, out_hbm, t0, d0a, bufs, gsems, wsems)

    @pl.when(jnp.logical_and(fast, jnp.logical_not(is_keys)))
    def _():
        d0a = pl.multiple_of(d0, 8)
        _fast_slab(queue_hbm, out_hbm, d0a, d0a, bufs, gsems, wsems)

    @pl.when(jnp.logical_not(fast))
    def _():
        for j in range(SLAB // PC):
            tp = pl.multiple_of(t0 + j * PC, PC)
            dp = (p + tp) & MASK
            lin = jnp.logical_and(aligned, dp <= K_Q - PC)
            kp = tp < B_K

            @pl.when(jnp.logical_and(lin, kp))
            def _():
                dpa = pl.multiple_of(dp, 8)
                pltpu.sync_copy(keys_hbm.at[pl.ds(tp, PC)], buf_v)
                pltpu.sync_copy(buf_v, out_hbm.at[pl.ds(dpa, PC)])

            @pl.when(jnp.logical_and(lin, jnp.logical_not(kp)))
            def _():
                dpa = pl.multiple_of(dp, 8)
                pltpu.sync_copy(queue_hbm.at[pl.ds(dpa, PC)], buf_v)
                pltpu.sync_copy(buf_v, out_hbm.at[pl.ds(dpa, PC)])

            @pl.when(jnp.logical_not(lin))
            def _():
                for q in range(PC // 16):
                    idx_v[pl.ds(q * 16, 16)] = (
                        dp + q * 16 + lax.iota(jnp.int32, 16)) & MASK

                @pl.when(kp)
                def _():
                    pltpu.sync_copy(keys_hbm.at[pl.ds(tp, PC)], buf_v)

                @pl.when(jnp.logical_not(kp))
                def _():
                    pltpu.async_copy(queue_hbm.at[idx_v], buf_v, sga).wait()

                pltpu.async_copy(buf_v, out_hbm.at[idx_v], sga).wait()


def kernel(keys, queue, ptr):
    mesh = plsc.VectorSubcoreMesh(core_axis_name="c", subcore_axis_name="s")
    run = functools.partial(
        pl.kernel,
        out_type=jax.ShapeDtypeStruct((K_Q, D), jnp.float32),
        mesh=mesh,
        scratch_types=[
            pltpu.VMEM((16,), jnp.int32),
            pltpu.VMEM((PC,), jnp.int32),
            pltpu.VMEM((PC, D), jnp.float32),
            pltpu.VMEM((CH, D), jnp.float32),
            pltpu.VMEM((CH, D), jnp.float32),
            pltpu.VMEM((CH, D), jnp.float32),
            pltpu.SemaphoreType.DMA,
            pltpu.SemaphoreType.DMA,
            pltpu.SemaphoreType.DMA,
            pltpu.SemaphoreType.DMA,
            pltpu.SemaphoreType.DMA,
            pltpu.SemaphoreType.DMA,
        ],
    )(_sc_body)
    return run(keys, queue, ptr.astype(jnp.int32))


# speedup vs baseline: 1.2465x; 1.0565x over previous
import functools

import jax
import jax.numpy as jnp
from jax import lax
from jax.experimental import pallas as pl
from jax.experimental.pallas import tpu as pltpu
from jax.experimental.pallas import tpu_sc as plsc

K_Q = 65536
D = 128
B_K = 4096
MASK = K_Q - 1
NC, NS = 2, 16
NW = NC * NS
PC = B_K // NW


def _sc_body(keys_hbm, ptr_hbm, out_hbm, ptr_v, idx_v, buf_v, sem):
    w = lax.axis_index("s") * NC + lax.axis_index("c")
    tp = pl.multiple_of(w * PC, PC)
    pltpu.sync_copy(ptr_hbm, ptr_v.at[pl.ds(0, 1)])
    p = ptr_v[...][0]
    dp = (p + tp) & MASK
    lin = jnp.logical_and((p & 7) == 0, dp <= K_Q - PC)

    pltpu.sync_copy(keys_hbm.at[pl.ds(tp, PC)], buf_v)

    @pl.when(lin)
    def _():
        dpa = pl.multiple_of(dp, 8)
        pltpu.sync_copy(buf_v, out_hbm.at[pl.ds(dpa, PC)])

    @pl.when(jnp.logical_not(lin))
    def _():
        for q in range(PC // 16):
            idx_v[pl.ds(q * 16, 16)] = (
                dp + q * 16 + lax.iota(jnp.int32, 16)) & MASK
        pltpu.async_copy(buf_v, out_hbm.at[idx_v], sem).wait()


def kernel(keys, queue, ptr):
    mesh = plsc.VectorSubcoreMesh(core_axis_name="c", subcore_axis_name="s")
    enqueue = functools.partial(
        pl.kernel,
        mesh=mesh,
        scratch_types=[
            pltpu.VMEM((16,), jnp.int32),
            pltpu.VMEM((PC,), jnp.int32),
            pltpu.VMEM((PC, D), jnp.float32),
            pltpu.SemaphoreType.DMA,
        ],
    )(_sc_body)
    out_ref = jax.new_ref(queue)
    enqueue(keys, ptr.astype(jnp.int32), out_ref)
    return out_ref[...]


# speedup vs baseline: 1.2694x; 1.0183x over previous
import functools

import jax
import jax.numpy as jnp
from jax import lax
from jax.experimental import pallas as pl
from jax.experimental.pallas import tpu as pltpu
from jax.experimental.pallas import tpu_sc as plsc

K_Q = 65536
D = 128
B_K = 4096
MASK = K_Q - 1
NC, NS = 1, 16
NW = NC * NS
PC = B_K // NW


def _sc_body(keys_hbm, ptr_hbm, out_hbm, ptr_v, idx_v, buf_v, sem):
    w = lax.axis_index("s") * NC + lax.axis_index("c")
    tp = pl.multiple_of(w * PC, PC)
    pltpu.sync_copy(ptr_hbm, ptr_v.at[pl.ds(0, 1)])
    p = ptr_v[...][0]
    dp = (p + tp) & MASK
    lin = jnp.logical_and((p & 7) == 0, dp <= K_Q - PC)

    pltpu.sync_copy(keys_hbm.at[pl.ds(tp, PC)], buf_v)

    @pl.when(lin)
    def _():
        dpa = pl.multiple_of(dp, 8)
        pltpu.sync_copy(buf_v, out_hbm.at[pl.ds(dpa, PC)])

    @pl.when(jnp.logical_not(lin))
    def _():
        for q in range(PC // 16):
            idx_v[pl.ds(q * 16, 16)] = (
                dp + q * 16 + lax.iota(jnp.int32, 16)) & MASK
        pltpu.async_copy(buf_v, out_hbm.at[idx_v], sem).wait()


def kernel(keys, queue, ptr):
    mesh = plsc.VectorSubcoreMesh(core_axis_name="c", subcore_axis_name="s", num_cores=1)
    enqueue = functools.partial(
        pl.kernel,
        mesh=mesh,
        scratch_types=[
            pltpu.VMEM((16,), jnp.int32),
            pltpu.VMEM((PC,), jnp.int32),
            pltpu.VMEM((PC, D), jnp.float32),
            pltpu.SemaphoreType.DMA,
        ],
    )(_sc_body)
    out_ref = jax.new_ref(queue)
    enqueue(keys, ptr.astype(jnp.int32), out_ref)
    return out_ref[...]
